# Initial kernel scaffold; baseline (speedup 1.0000x reference)
import jax
import jax.numpy as jnp
from jax.experimental import pallas as pl


def kernel(x, edge_index, edge_attr, W1, b1, gamma1, beta1, W3, b3):
    raise NotImplementedError("write your pallas kernel here")


# speedup vs baseline: 66.0746x; 66.0746x over previous
import functools

import jax
import jax.numpy as jnp
from jax import lax
from jax.experimental import pallas as pl
from jax.experimental.pallas import tpu as pltpu
from jax.experimental.pallas import tpu_sc as plsc

_N = 100000
_E = 1600000
_EPS = 1e-5

_NC = 2
_NS = 16
_NW = _NC * _NS

_NPAD = 102400
_TSL = _NPAD // _NS
_ROWS = 12544
_EPAD = _ROWS * 128
_RW = _ROWS // _NW
_K = 56
_NWIN = _RW // _K


def _worker_ids():
  cid = lax.axis_index("c")
  sid = lax.axis_index("s")
  return cid, sid, sid * _NC + cid


def _fill(buf, value, n16):
  def body(i, carry):
    buf[pl.ds(i * 16, 16)] = jnp.zeros((16,), jnp.float32) + value
    return carry
  lax.fori_loop(0, n16, body, 0)


def _sc_deg_body(col_hbm, w_hbm, degp_hbm, idxv, wv, zbuf, deg_sh):
  cid, sid, wid = _worker_ids()
  sl = pl.ds(sid * _TSL, _TSL)
  initv = jnp.where(cid == 0, 1.0, 0.0)
  _fill(zbuf, initv, _TSL // 16)
  pltpu.sync_copy(zbuf, deg_sh.at[sl])
  plsc.subcore_barrier()

  base = wid * _RW

  def window(t, carry):
    wb = base + t * _K
    pltpu.sync_copy(col_hbm.at[pl.ds(wb, _K)], idxv)
    pltpu.sync_copy(w_hbm.at[pl.ds(wb, _K)], wv)

    def row(j, c2):
      pltpu.sync_copy(wv.at[j], deg_sh.at[idxv.at[j]], add=True)
      return c2
    lax.fori_loop(0, _K, row, 0)
    return carry
  lax.fori_loop(0, _NWIN, window, 0)

  plsc.subcore_barrier()
  pltpu.sync_copy(deg_sh.at[sl],
                  degp_hbm.at[pl.ds(cid * _NPAD + sid * _TSL, _TSL)])


_sc_deg = functools.partial(
    pl.kernel,
    out_type=jax.ShapeDtypeStruct((_NC * _NPAD,), jnp.float32),
    mesh=plsc.VectorSubcoreMesh(core_axis_name="c", subcore_axis_name="s"),
    scratch_types=[
        pltpu.VMEM((_K, 128), jnp.int32),
        pltpu.VMEM((_K, 128), jnp.float32),
        pltpu.VMEM((_TSL,), jnp.float32),
        pltpu.VMEM_SHARED((_NPAD,), jnp.float32),
    ],
)(_sc_deg_body)


def _sc_agg_body(row_hbm, col_hbm, w_hbm, dinv_hbm, x0_hbm, x1_hbm, x2_hbm,
                 aggp_hbm, norm_hbm,
                 idxr, idxc, wv, drv, dcv, nv, xg, mg, dbuf, xbuf,
                 dinv_sh, x0_sh, x1_sh, x2_sh, a0_sh, a1_sh, a2_sh):
  cid, sid, wid = _worker_ids()
  sl = pl.ds(sid * _TSL, _TSL)
  x_hbms = (x0_hbm, x1_hbm, x2_hbm)
  x_shs = (x0_sh, x1_sh, x2_sh)
  a_shs = (a0_sh, a1_sh, a2_sh)

  pltpu.sync_copy(dinv_hbm.at[sl], dbuf)
  pltpu.sync_copy(dbuf, dinv_sh.at[sl])
  initm = jnp.where(cid == 0, 1.0, 0.0)
  for c in range(3):
    pltpu.sync_copy(x_hbms[c].at[sl], xbuf)
    pltpu.sync_copy(xbuf, x_shs[c].at[sl])

    def finit(i, carry):
      s = pl.ds(i * 16, 16)
      xbuf[s] = xbuf[s] * dbuf[s] * dbuf[s] * initm
      return carry
    lax.fori_loop(0, _TSL // 16, finit, 0)
    pltpu.sync_copy(xbuf, a_shs[c].at[sl])
  plsc.subcore_barrier()

  base = wid * _RW

  def window(t, carry):
    wb = base + t * _K
    pltpu.sync_copy(row_hbm.at[pl.ds(wb, _K)], idxr)
    pltpu.sync_copy(col_hbm.at[pl.ds(wb, _K)], idxc)
    pltpu.sync_copy(w_hbm.at[pl.ds(wb, _K)], wv)

    def g_dinv(j, c2):
      pltpu.sync_copy(dinv_sh.at[idxr.at[j]], drv.at[j])
      pltpu.sync_copy(dinv_sh.at[idxc.at[j]], dcv.at[j])
      return c2
    lax.fori_loop(0, _K, g_dinv, 0)

    def v_norm(j, c2):
      def vm(m, c3):
        s = pl.ds(m * 16, 16)
        nv[j, s] = drv[j, s] * wv[j, s] * dcv[j, s]
        return c3
      lax.fori_loop(0, 8, vm, 0)
      return c2
    lax.fori_loop(0, _K, v_norm, 0)
    pltpu.sync_copy(nv, norm_hbm.at[pl.ds(wb, _K)])

    for c in range(3):
      def g_x(j, c2):
        pltpu.sync_copy(x_shs[c].at[idxr.at[j]], xg.at[j])
        return c2
      lax.fori_loop(0, _K, g_x, 0)

      def v_mul(j, c2):
        def vm(m, c3):
          s = pl.ds(m * 16, 16)
          mg[j, s] = xg[j, s] * nv[j, s]
          return c3
        lax.fori_loop(0, 8, vm, 0)
        return c2
      lax.fori_loop(0, _K, v_mul, 0)

      def s_add(j, c2):
        pltpu.sync_copy(mg.at[j], a_shs[c].at[idxc.at[j]], add=True)
        return c2
      lax.fori_loop(0, _K, s_add, 0)
    return carry
  lax.fori_loop(0, _NWIN, window, 0)

  plsc.subcore_barrier()
  for c in range(3):
    off = (cid * 3 + c) * _NPAD + sid * _TSL
    pltpu.sync_copy(a_shs[c].at[sl], aggp_hbm.at[pl.ds(off, _TSL)])


_sc_agg = functools.partial(
    pl.kernel,
    out_type=[
        jax.ShapeDtypeStruct((_NC * 3 * _NPAD,), jnp.float32),
        jax.ShapeDtypeStruct((_ROWS, 128), jnp.float32),
    ],
    mesh=plsc.VectorSubcoreMesh(core_axis_name="c", subcore_axis_name="s"),
    scratch_types=[
        pltpu.VMEM((_K, 128), jnp.int32),
        pltpu.VMEM((_K, 128), jnp.int32),
        pltpu.VMEM((_K, 128), jnp.float32),
        pltpu.VMEM((_K, 128), jnp.float32),
        pltpu.VMEM((_K, 128), jnp.float32),
        pltpu.VMEM((_K, 128), jnp.float32),
        pltpu.VMEM((_K, 128), jnp.float32),
        pltpu.VMEM((_K, 128), jnp.float32),
        pltpu.VMEM((_TSL,), jnp.float32),
        pltpu.VMEM((_TSL,), jnp.float32),
        pltpu.VMEM_SHARED((_NPAD,), jnp.float32),
        pltpu.VMEM_SHARED((_NPAD,), jnp.float32),
        pltpu.VMEM_SHARED((_NPAD,), jnp.float32),
        pltpu.VMEM_SHARED((_NPAD,), jnp.float32),
        pltpu.VMEM_SHARED((_NPAD,), jnp.float32),
        pltpu.VMEM_SHARED((_NPAD,), jnp.float32),
        pltpu.VMEM_SHARED((_NPAD,), jnp.float32),
    ],
)(_sc_agg_body)


def _sc_out_body(row_hbm, col_hbm, norm_hbm, z_hbm, outp_hbm,
                 idxr, idxc, nv, zg, mg, zbuf, z_sh, o_sh):
  cid, sid, wid = _worker_ids()
  sl = pl.ds(sid * _TSL, _TSL)
  pltpu.sync_copy(z_hbm.at[sl], zbuf)
  pltpu.sync_copy(zbuf, z_sh.at[sl])
  _fill(zbuf, 0.0, _TSL // 16)
  pltpu.sync_copy(zbuf, o_sh.at[sl])
  plsc.subcore_barrier()

  base = wid * _RW

  def window(t, carry):
    wb = base + t * _K
    pltpu.sync_copy(row_hbm.at[pl.ds(wb, _K)], idxr)
    pltpu.sync_copy(col_hbm.at[pl.ds(wb, _K)], idxc)
    pltpu.sync_copy(norm_hbm.at[pl.ds(wb, _K)], nv)

    def g_z(j, c2):
      pltpu.sync_copy(z_sh.at[idxr.at[j]], zg.at[j])
      return c2
    lax.fori_loop(0, _K, g_z, 0)

    def v_mul(j, c2):
      def vm(m, c3):
        s = pl.ds(m * 16, 16)
        mg[j, s] = zg[j, s] * nv[j, s]
        return c3
      lax.fori_loop(0, 8, vm, 0)
      return c2
    lax.fori_loop(0, _K, v_mul, 0)

    def s_add(j, c2):
      pltpu.sync_copy(mg.at[j], o_sh.at[idxc.at[j]], add=True)
      return c2
    lax.fori_loop(0, _K, s_add, 0)
    return carry
  lax.fori_loop(0, _NWIN, window, 0)

  plsc.subcore_barrier()
  pltpu.sync_copy(o_sh.at[sl],
                  outp_hbm.at[pl.ds(cid * _NPAD + sid * _TSL, _TSL)])


_sc_out = functools.partial(
    pl.kernel,
    out_type=jax.ShapeDtypeStruct((_NC * _NPAD,), jnp.float32),
    mesh=plsc.VectorSubcoreMesh(core_axis_name="c", subcore_axis_name="s"),
    scratch_types=[
        pltpu.VMEM((_K, 128), jnp.int32),
        pltpu.VMEM((_K, 128), jnp.int32),
        pltpu.VMEM((_K, 128), jnp.float32),
        pltpu.VMEM((_K, 128), jnp.float32),
        pltpu.VMEM((_K, 128), jnp.float32),
        pltpu.VMEM((_TSL,), jnp.float32),
        pltpu.VMEM_SHARED((_NPAD,), jnp.float32),
        pltpu.VMEM_SHARED((_NPAD,), jnp.float32),
    ],
)(_sc_out_body)


def _tc_dinv_body(degp_ref, o_ref):
  o_ref[...] = lax.rsqrt(degp_ref[0] + degp_ref[1])


def _tc_z_body(aggp_ref, w1_ref, b1_ref, g1_ref, be1_ref, w3_ref, z_ref):
  a = [aggp_ref[0, c] + aggp_ref[1, c] for c in range(3)]
  nn = float(_N)
  s1 = [jnp.sum(a[c]) / nn for c in range(3)]
  w1r = [w1_ref[c] for c in range(3)]
  b1 = b1_ref[...]
  mean = s1[0] * w1r[0] + s1[1] * w1r[1] + s1[2] * w1r[2] + b1
  var = jnp.zeros((32,), jnp.float32)
  for k in range(3):
    for l in range(3):
      ckl = jnp.sum(a[k] * a[l]) / nn - s1[k] * s1[l]
      var = var + ckl * w1r[k] * w1r[l]
  istd = g1_ref[...] * lax.rsqrt(var + _EPS)
  wf = [w1r[c] * istd for c in range(3)]
  cf = (b1 - mean) * istd + be1_ref[...]
  w3 = w3_ref[...][:, 0]
  acc = jnp.zeros((_NPAD // 128, 128), jnp.float32)
  for j in range(32):
    h = a[0] * wf[0][j] + a[1] * wf[1][j] + a[2] * wf[2][j] + cf[j]
    acc = acc + jnp.maximum(h, 0.0) * w3[j]
  z_ref[...] = acc


def _tc_fin_body(outp_ref, dinv_ref, z_ref, b3_ref, o_ref):
  d = dinv_ref[...]
  o_ref[...] = (outp_ref[0] + outp_ref[1] + d * d * z_ref[...]
                + b3_ref[0, 0])


def kernel(x, edge_index, edge_attr, W1, b1, gamma1, beta1, W3, b3):
  f32 = jnp.float32
  row = edge_index[0].astype(jnp.int32)
  col = edge_index[1].astype(jnp.int32)
  w = edge_attr.astype(f32)

  npad_e = _EPAD - _E
  pad_idx = (_N + (jnp.arange(npad_e, dtype=jnp.int32) % (_NPAD - _N)))
  row_p = jnp.concatenate([row, pad_idx]).reshape(_ROWS, 128)
  col_p = jnp.concatenate([col, pad_idx]).reshape(_ROWS, 128)
  w_p = jnp.concatenate([w, jnp.zeros((npad_e,), f32)]).reshape(_ROWS, 128)

  xt = jnp.pad(x.astype(f32), ((0, _NPAD - _N), (0, 0))).T
  x0, x1, x2 = xt[0], xt[1], xt[2]

  degp = _sc_deg(col_p, w_p)

  dinv2d = pl.pallas_call(
      _tc_dinv_body,
      out_shape=jax.ShapeDtypeStruct((_NPAD // 128, 128), f32),
  )(degp.reshape(_NC, _NPAD // 128, 128))
  dinv = dinv2d.reshape(_NPAD)

  aggp, norm2d = _sc_agg(row_p, col_p, w_p, dinv, x0, x1, x2)

  z2d = pl.pallas_call(
      _tc_z_body,
      out_shape=jax.ShapeDtypeStruct((_NPAD // 128, 128), f32),
  )(aggp.reshape(_NC, 3, _NPAD // 128, 128), W1.astype(f32),
    b1.astype(f32), gamma1.astype(f32), beta1.astype(f32), W3.astype(f32))
  z = z2d.reshape(_NPAD)

  outp = _sc_out(row_p, col_p, norm2d, z)

  out2d = pl.pallas_call(
      _tc_fin_body,
      out_shape=jax.ShapeDtypeStruct((_NPAD // 128, 128), f32),
  )(outp.reshape(_NC, _NPAD // 128, 128), dinv2d, z2d,
    b3.astype(f32).reshape(1, 1))

  return out2d.reshape(_NPAD)[:_N, None]


# speedup vs baseline: 127.8950x; 1.9356x over previous
import functools

import jax
import jax.numpy as jnp
from jax import lax
from jax.experimental import pallas as pl
from jax.experimental.pallas import tpu as pltpu
from jax.experimental.pallas import tpu_sc as plsc

_N = 100000
_E = 1600000
_EPS = 1e-5

_NC = 2
_NS = 16
_NW = _NC * _NS

_NPAD = 102400
_TSL = _NPAD // _NS
_ROWS = 12800
_EPAD = _ROWS * 128
_RW = _ROWS // _NW
_K = 40
_NWIN = _RW // _K


def _worker_ids():
  cid = lax.axis_index("c")
  sid = lax.axis_index("s")
  return cid, sid, sid * _NC + cid


def _fill(buf, value, n16):
  def body(i, carry):
    buf[pl.ds(i * 16, 16)] = jnp.zeros((16,), jnp.float32) + value
    return carry
  lax.fori_loop(0, n16, body, 0)


def _sc_deg_body(col_hbm, w_hbm, degp_hbm, idxc, wv, zbuf, deg_sh, sem):
  cid, sid, wid = _worker_ids()
  sl = pl.ds(sid * _TSL, _TSL)
  initv = jnp.where(cid == 0, 1.0, 0.0)
  _fill(zbuf, initv, _TSL // 16)
  pltpu.sync_copy(zbuf, deg_sh.at[sl])
  plsc.subcore_barrier()

  base = wid * _RW

  def window(t, carry):
    wb = base + t * _K
    pltpu.sync_copy(col_hbm.at[pl.ds(wb, _K)], idxc)
    pltpu.sync_copy(w_hbm.at[pl.ds(wb, _K)], wv)

    descs = [
        pltpu.async_copy(wv.at[j], deg_sh.at[idxc.at[j]], sem, add=True)
        for j in range(_K)
    ]
    for d in descs:
      d.wait()
    return carry
  lax.fori_loop(0, _NWIN, window, 0)

  plsc.subcore_barrier()
  pltpu.sync_copy(deg_sh.at[sl],
                  degp_hbm.at[pl.ds(cid * _NPAD + sid * _TSL, _TSL)])


_sc_deg = functools.partial(
    pl.kernel,
    out_type=jax.ShapeDtypeStruct((_NC * _NPAD,), jnp.float32),
    mesh=plsc.VectorSubcoreMesh(core_axis_name="c", subcore_axis_name="s"),
    scratch_types=[
        pltpu.VMEM((_K, 128), jnp.int32),
        pltpu.VMEM((_K, 128), jnp.float32),
        pltpu.VMEM((_TSL,), jnp.float32),
        pltpu.VMEM_SHARED((_NPAD,), jnp.float32),
        pltpu.SemaphoreType.DMA,
    ],
)(_sc_deg_body)


def _sc_agg_body(row_hbm, col_hbm, w_hbm, dinv_hbm, x0_hbm, x1_hbm, x2_hbm,
                 aggp_hbm, norm_hbm,
                 idxr, idxc, wv, drv, dcv, xg0, xg1, xg2, dbuf, xbuf,
                 dinv_sh, x0_sh, x1_sh, x2_sh, a0_sh, a1_sh, a2_sh, sem):
  cid, sid, wid = _worker_ids()
  sl = pl.ds(sid * _TSL, _TSL)
  x_hbms = (x0_hbm, x1_hbm, x2_hbm)
  x_shs = (x0_sh, x1_sh, x2_sh)
  a_shs = (a0_sh, a1_sh, a2_sh)
  xgs = (xg0, xg1, xg2)

  pltpu.sync_copy(dinv_hbm.at[sl], dbuf)
  pltpu.sync_copy(dbuf, dinv_sh.at[sl])
  initm = jnp.where(cid == 0, 1.0, 0.0)
  for c in range(3):
    pltpu.sync_copy(x_hbms[c].at[sl], xbuf)
    pltpu.sync_copy(xbuf, x_shs[c].at[sl])

    def finit(i, carry):
      s = pl.ds(i * 16, 16)
      xbuf[s] = xbuf[s] * dbuf[s] * dbuf[s] * initm
      return carry
    lax.fori_loop(0, _TSL // 16, finit, 0)
    pltpu.sync_copy(xbuf, a_shs[c].at[sl])
  plsc.subcore_barrier()

  base = wid * _RW

  def window(t, carry):
    wb = base + t * _K
    pltpu.sync_copy(row_hbm.at[pl.ds(wb, _K)], idxr)
    pltpu.sync_copy(col_hbm.at[pl.ds(wb, _K)], idxc)
    pltpu.sync_copy(w_hbm.at[pl.ds(wb, _K)], wv)

    gds = [pltpu.async_copy(dinv_sh.at[idxr.at[j]], drv.at[j], sem)
           for j in range(_K)]
    gds += [pltpu.async_copy(dinv_sh.at[idxc.at[j]], dcv.at[j], sem)
            for j in range(_K)]
    for d in gds:
      d.wait()
    gds = []
    for c in range(3):
      gds += [pltpu.async_copy(x_shs[c].at[idxr.at[j]], xgs[c].at[j], sem)
              for j in range(_K)]

    def v_norm(j, c2):
      def vm(m, c3):
        s = pl.ds(m * 16, 16)
        wv[j, s] = drv[j, s] * wv[j, s] * dcv[j, s]
        return c3
      lax.fori_loop(0, 8, vm, 0)
      return c2
    lax.fori_loop(0, _K, v_norm, 0)
    for d in gds:
      d.wait()
    nd = pltpu.async_copy(wv, norm_hbm.at[pl.ds(wb, _K)], sem)

    def v_mul(j, c2):
      def vm(m, c3):
        s = pl.ds(m * 16, 16)
        n = wv[j, s]
        xg0[j, s] = xg0[j, s] * n
        xg1[j, s] = xg1[j, s] * n
        xg2[j, s] = xg2[j, s] * n
        return c3
      lax.fori_loop(0, 8, vm, 0)
      return c2
    lax.fori_loop(0, _K, v_mul, 0)

    descs = []
    for c in range(3):
      descs += [
          pltpu.async_copy(xgs[c].at[j], a_shs[c].at[idxc.at[j]], sem,
                           add=True)
          for j in range(_K)
      ]
    nd.wait()
    for d in descs:
      d.wait()
    return carry
  lax.fori_loop(0, _NWIN, window, 0)

  plsc.subcore_barrier()
  for c in range(3):
    off = (cid * 3 + c) * _NPAD + sid * _TSL
    pltpu.sync_copy(a_shs[c].at[sl], aggp_hbm.at[pl.ds(off, _TSL)])


_sc_agg = functools.partial(
    pl.kernel,
    out_type=[
        jax.ShapeDtypeStruct((_NC * 3 * _NPAD,), jnp.float32),
        jax.ShapeDtypeStruct((_ROWS, 128), jnp.float32),
    ],
    mesh=plsc.VectorSubcoreMesh(core_axis_name="c", subcore_axis_name="s"),
    scratch_types=[
        pltpu.VMEM((_K, 128), jnp.int32),
        pltpu.VMEM((_K, 128), jnp.int32),
        pltpu.VMEM((_K, 128), jnp.float32),
        pltpu.VMEM((_K, 128), jnp.float32),
        pltpu.VMEM((_K, 128), jnp.float32),
        pltpu.VMEM((_K, 128), jnp.float32),
        pltpu.VMEM((_K, 128), jnp.float32),
        pltpu.VMEM((_K, 128), jnp.float32),
        pltpu.VMEM((_TSL,), jnp.float32),
        pltpu.VMEM((_TSL,), jnp.float32),
        pltpu.VMEM_SHARED((_NPAD,), jnp.float32),
        pltpu.VMEM_SHARED((_NPAD,), jnp.float32),
        pltpu.VMEM_SHARED((_NPAD,), jnp.float32),
        pltpu.VMEM_SHARED((_NPAD,), jnp.float32),
        pltpu.VMEM_SHARED((_NPAD,), jnp.float32),
        pltpu.VMEM_SHARED((_NPAD,), jnp.float32),
        pltpu.VMEM_SHARED((_NPAD,), jnp.float32),
        pltpu.SemaphoreType.DMA,
    ],
)(_sc_agg_body)


def _sc_out_body(row_hbm, col_hbm, norm_hbm, z_hbm, outp_hbm,
                 idxr, idxc, nv, zg, zbuf, z_sh, o_sh, sem):
  cid, sid, wid = _worker_ids()
  sl = pl.ds(sid * _TSL, _TSL)
  pltpu.sync_copy(z_hbm.at[sl], zbuf)
  pltpu.sync_copy(zbuf, z_sh.at[sl])
  _fill(zbuf, 0.0, _TSL // 16)
  pltpu.sync_copy(zbuf, o_sh.at[sl])
  plsc.subcore_barrier()

  base = wid * _RW

  def window(t, carry):
    wb = base + t * _K
    pltpu.sync_copy(row_hbm.at[pl.ds(wb, _K)], idxr)
    pltpu.sync_copy(col_hbm.at[pl.ds(wb, _K)], idxc)
    pltpu.sync_copy(norm_hbm.at[pl.ds(wb, _K)], nv)

    gds = [pltpu.async_copy(z_sh.at[idxr.at[j]], zg.at[j], sem)
           for j in range(_K)]
    for d in gds:
      d.wait()

    def v_mul(j, c2):
      def vm(m, c3):
        s = pl.ds(m * 16, 16)
        zg[j, s] = zg[j, s] * nv[j, s]
        return c3
      lax.fori_loop(0, 8, vm, 0)
      return c2
    lax.fori_loop(0, _K, v_mul, 0)

    descs = [
        pltpu.async_copy(zg.at[j], o_sh.at[idxc.at[j]], sem, add=True)
        for j in range(_K)
    ]
    for d in descs:
      d.wait()
    return carry
  lax.fori_loop(0, _NWIN, window, 0)

  plsc.subcore_barrier()
  pltpu.sync_copy(o_sh.at[sl],
                  outp_hbm.at[pl.ds(cid * _NPAD + sid * _TSL, _TSL)])


_sc_out = functools.partial(
    pl.kernel,
    out_type=jax.ShapeDtypeStruct((_NC * _NPAD,), jnp.float32),
    mesh=plsc.VectorSubcoreMesh(core_axis_name="c", subcore_axis_name="s"),
    scratch_types=[
        pltpu.VMEM((_K, 128), jnp.int32),
        pltpu.VMEM((_K, 128), jnp.int32),
        pltpu.VMEM((_K, 128), jnp.float32),
        pltpu.VMEM((_K, 128), jnp.float32),
        pltpu.VMEM((_TSL,), jnp.float32),
        pltpu.VMEM_SHARED((_NPAD,), jnp.float32),
        pltpu.VMEM_SHARED((_NPAD,), jnp.float32),
        pltpu.SemaphoreType.DMA,
    ],
)(_sc_out_body)


def _tc_dinv_body(degp_ref, o_ref):
  o_ref[...] = lax.rsqrt(degp_ref[0] + degp_ref[1])


def _tc_z_body(aggp_ref, w1_ref, b1_ref, g1_ref, be1_ref, w3_ref, z_ref):
  a = [aggp_ref[0, c] + aggp_ref[1, c] for c in range(3)]
  nn = float(_N)
  s1 = [jnp.sum(a[c]) / nn for c in range(3)]
  w1r = [w1_ref[c] for c in range(3)]
  b1 = b1_ref[...]
  mean = s1[0] * w1r[0] + s1[1] * w1r[1] + s1[2] * w1r[2] + b1
  var = jnp.zeros((32,), jnp.float32)
  for k in range(3):
    for l in range(3):
      ckl = jnp.sum(a[k] * a[l]) / nn - s1[k] * s1[l]
      var = var + ckl * w1r[k] * w1r[l]
  istd = g1_ref[...] * lax.rsqrt(var + _EPS)
  wf = [w1r[c] * istd for c in range(3)]
  cf = (b1 - mean) * istd + be1_ref[...]
  w3 = w3_ref[...][:, 0]
  acc = jnp.zeros((_NPAD // 128, 128), jnp.float32)
  for j in range(32):
    h = a[0] * wf[0][j] + a[1] * wf[1][j] + a[2] * wf[2][j] + cf[j]
    acc = acc + jnp.maximum(h, 0.0) * w3[j]
  z_ref[...] = acc


def _tc_fin_body(outp_ref, dinv_ref, z_ref, b3_ref, o_ref):
  d = dinv_ref[...]
  o_ref[...] = (outp_ref[0] + outp_ref[1] + d * d * z_ref[...]
                + b3_ref[0, 0])


def kernel(x, edge_index, edge_attr, W1, b1, gamma1, beta1, W3, b3):
  f32 = jnp.float32
  row = edge_index[0].astype(jnp.int32)
  col = edge_index[1].astype(jnp.int32)
  w = edge_attr.astype(f32)

  npad_e = _EPAD - _E
  pad_idx = (_N + (jnp.arange(npad_e, dtype=jnp.int32) % (_NPAD - _N)))
  row_p = jnp.concatenate([row, pad_idx]).reshape(_ROWS, 128)
  col_p = jnp.concatenate([col, pad_idx]).reshape(_ROWS, 128)
  w_p = jnp.concatenate([w, jnp.zeros((npad_e,), f32)]).reshape(_ROWS, 128)

  xt = jnp.pad(x.astype(f32), ((0, _NPAD - _N), (0, 0))).T
  x0, x1, x2 = xt[0], xt[1], xt[2]

  degp = _sc_deg(col_p, w_p)

  dinv2d = pl.pallas_call(
      _tc_dinv_body,
      out_shape=jax.ShapeDtypeStruct((_NPAD // 128, 128), f32),
  )(degp.reshape(_NC, _NPAD // 128, 128))
  dinv = dinv2d.reshape(_NPAD)

  aggp, norm2d = _sc_agg(row_p, col_p, w_p, dinv, x0, x1, x2)

  z2d = pl.pallas_call(
      _tc_z_body,
      out_shape=jax.ShapeDtypeStruct((_NPAD // 128, 128), f32),
  )(aggp.reshape(_NC, 3, _NPAD // 128, 128), W1.astype(f32),
    b1.astype(f32), gamma1.astype(f32), beta1.astype(f32), W3.astype(f32))
  z = z2d.reshape(_NPAD)

  outp = _sc_out(row_p, col_p, norm2d, z)

  out2d = pl.pallas_call(
      _tc_fin_body,
      out_shape=jax.ShapeDtypeStruct((_NPAD // 128, 128), f32),
  )(outp.reshape(_NC, _NPAD // 128, 128), dinv2d, z2d,
    b3.astype(f32).reshape(1, 1))

  return out2d.reshape(_NPAD)[:_N, None]


# speedup vs baseline: 140.2316x; 1.0965x over previous
import functools

import jax
import jax.numpy as jnp
from jax import lax
from jax.experimental import pallas as pl
from jax.experimental.pallas import tpu as pltpu
from jax.experimental.pallas import tpu_sc as plsc

_N = 100000
_E = 1600000
_EPS = 1e-5

_NC = 2
_NS = 16
_NW = _NC * _NS

_NPAD = 102400
_TSL = _NPAD // _NS
_ROWS = 12800
_EPAD = _ROWS * 128
_RW = _ROWS // _NW
_K = 40
_NWIN = _RW // _K


def _worker_ids():
  cid = lax.axis_index("c")
  sid = lax.axis_index("s")
  return cid, sid, sid * _NC + cid


def _fill(buf, value, n16):
  def body(i, carry):
    buf[pl.ds(i * 16, 16)] = jnp.zeros((16,), jnp.float32) + value
    return carry
  lax.fori_loop(0, n16, body, 0)


def _sc_deg_body(col_hbm, w_hbm, degp_hbm, idxc, wv, zbuf, deg_sh, sem):
  cid, sid, wid = _worker_ids()
  sl = pl.ds(sid * _TSL, _TSL)
  initv = jnp.where(cid == 0, 1.0, 0.0)
  _fill(zbuf, initv, _TSL // 16)
  pltpu.sync_copy(zbuf, deg_sh.at[sl])
  plsc.subcore_barrier()

  base = wid * _RW

  def window(t, carry):
    wb = base + t * _K
    pltpu.sync_copy(col_hbm.at[pl.ds(wb, _K)], idxc)
    pltpu.sync_copy(w_hbm.at[pl.ds(wb, _K)], wv)

    descs = [
        pltpu.async_copy(wv.at[j], deg_sh.at[idxc.at[j]], sem, add=True)
        for j in range(_K)
    ]
    for d in descs:
      d.wait()
    return carry
  lax.fori_loop(0, _NWIN, window, 0)

  plsc.subcore_barrier()
  pltpu.sync_copy(deg_sh.at[sl],
                  degp_hbm.at[pl.ds(cid * _NPAD + sid * _TSL, _TSL)])


_sc_deg = functools.partial(
    pl.kernel,
    out_type=jax.ShapeDtypeStruct((_NC * _NPAD,), jnp.float32),
    mesh=plsc.VectorSubcoreMesh(core_axis_name="c", subcore_axis_name="s"),
    scratch_types=[
        pltpu.VMEM((_K, 128), jnp.int32),
        pltpu.VMEM((_K, 128), jnp.float32),
        pltpu.VMEM((_TSL,), jnp.float32),
        pltpu.VMEM_SHARED((_NPAD,), jnp.float32),
        pltpu.SemaphoreType.DMA,
    ],
)(_sc_deg_body)


def _sc_agg_body(row_hbm, col_hbm, w_hbm, y0_hbm, y1_hbm, y2_hbm,
                 aggp_hbm,
                 idxr, idxc, wv, yg0, yg1, yg2, buf,
                 y0_sh, y1_sh, y2_sh, a0_sh, a1_sh, a2_sh, sem):
  cid, sid, wid = _worker_ids()
  sl = pl.ds(sid * _TSL, _TSL)
  y_hbms = (y0_hbm, y1_hbm, y2_hbm)
  y_shs = (y0_sh, y1_sh, y2_sh)
  a_shs = (a0_sh, a1_sh, a2_sh)
  ygs = (yg0, yg1, yg2)

  for c in range(3):
    pltpu.sync_copy(y_hbms[c].at[sl], buf)
    pltpu.sync_copy(buf, y_shs[c].at[sl])
  _fill(buf, 0.0, _TSL // 16)
  for c in range(3):
    pltpu.sync_copy(buf, a_shs[c].at[sl])
  plsc.subcore_barrier()

  base = wid * _RW

  def window(t, carry):
    wb = base + t * _K
    pltpu.sync_copy(row_hbm.at[pl.ds(wb, _K)], idxr)
    pltpu.sync_copy(col_hbm.at[pl.ds(wb, _K)], idxc)
    pltpu.sync_copy(w_hbm.at[pl.ds(wb, _K)], wv)

    gds = []
    for c in range(3):
      gds += [pltpu.async_copy(y_shs[c].at[idxr.at[j]], ygs[c].at[j], sem)
              for j in range(_K)]
    for d in gds:
      d.wait()

    def v_mul(j, c2):
      def vm(m, c3):
        s = pl.ds(m * 16, 16)
        n = wv[j, s]
        yg0[j, s] = yg0[j, s] * n
        yg1[j, s] = yg1[j, s] * n
        yg2[j, s] = yg2[j, s] * n
        return c3
      lax.fori_loop(0, 8, vm, 0)
      return c2
    lax.fori_loop(0, _K, v_mul, 0)

    descs = []
    for c in range(3):
      descs += [
          pltpu.async_copy(ygs[c].at[j], a_shs[c].at[idxc.at[j]], sem,
                           add=True)
          for j in range(_K)
      ]
    for d in descs:
      d.wait()
    return carry
  lax.fori_loop(0, _NWIN, window, 0)

  plsc.subcore_barrier()
  for c in range(3):
    off = (cid * 3 + c) * _NPAD + sid * _TSL
    pltpu.sync_copy(a_shs[c].at[sl], aggp_hbm.at[pl.ds(off, _TSL)])


_sc_agg = functools.partial(
    pl.kernel,
    out_type=jax.ShapeDtypeStruct((_NC * 3 * _NPAD,), jnp.float32),
    mesh=plsc.VectorSubcoreMesh(core_axis_name="c", subcore_axis_name="s"),
    scratch_types=[
        pltpu.VMEM((_K, 128), jnp.int32),
        pltpu.VMEM((_K, 128), jnp.int32),
        pltpu.VMEM((_K, 128), jnp.float32),
        pltpu.VMEM((_K, 128), jnp.float32),
        pltpu.VMEM((_K, 128), jnp.float32),
        pltpu.VMEM((_K, 128), jnp.float32),
        pltpu.VMEM((_TSL,), jnp.float32),
        pltpu.VMEM_SHARED((_NPAD,), jnp.float32),
        pltpu.VMEM_SHARED((_NPAD,), jnp.float32),
        pltpu.VMEM_SHARED((_NPAD,), jnp.float32),
        pltpu.VMEM_SHARED((_NPAD,), jnp.float32),
        pltpu.VMEM_SHARED((_NPAD,), jnp.float32),
        pltpu.VMEM_SHARED((_NPAD,), jnp.float32),
        pltpu.SemaphoreType.DMA,
    ],
)(_sc_agg_body)


def _sc_out_body(row_hbm, col_hbm, w_hbm, zd_hbm, outp_hbm,
                 idxr, idxc, wv, zg, zbuf, z_sh, o_sh, sem):
  cid, sid, wid = _worker_ids()
  sl = pl.ds(sid * _TSL, _TSL)
  pltpu.sync_copy(zd_hbm.at[sl], zbuf)
  pltpu.sync_copy(zbuf, z_sh.at[sl])
  _fill(zbuf, 0.0, _TSL // 16)
  pltpu.sync_copy(zbuf, o_sh.at[sl])
  plsc.subcore_barrier()

  base = wid * _RW

  def window(t, carry):
    wb = base + t * _K
    pltpu.sync_copy(row_hbm.at[pl.ds(wb, _K)], idxr)
    pltpu.sync_copy(col_hbm.at[pl.ds(wb, _K)], idxc)
    pltpu.sync_copy(w_hbm.at[pl.ds(wb, _K)], wv)

    gds = [pltpu.async_copy(z_sh.at[idxr.at[j]], zg.at[j], sem)
           for j in range(_K)]
    for d in gds:
      d.wait()

    def v_mul(j, c2):
      def vm(m, c3):
        s = pl.ds(m * 16, 16)
        zg[j, s] = zg[j, s] * wv[j, s]
        return c3
      lax.fori_loop(0, 8, vm, 0)
      return c2
    lax.fori_loop(0, _K, v_mul, 0)

    descs = [
        pltpu.async_copy(zg.at[j], o_sh.at[idxc.at[j]], sem, add=True)
        for j in range(_K)
    ]
    for d in descs:
      d.wait()
    return carry
  lax.fori_loop(0, _NWIN, window, 0)

  plsc.subcore_barrier()
  pltpu.sync_copy(o_sh.at[sl],
                  outp_hbm.at[pl.ds(cid * _NPAD + sid * _TSL, _TSL)])


_sc_out = functools.partial(
    pl.kernel,
    out_type=jax.ShapeDtypeStruct((_NC * _NPAD,), jnp.float32),
    mesh=plsc.VectorSubcoreMesh(core_axis_name="c", subcore_axis_name="s"),
    scratch_types=[
        pltpu.VMEM((_K, 128), jnp.int32),
        pltpu.VMEM((_K, 128), jnp.int32),
        pltpu.VMEM((_K, 128), jnp.float32),
        pltpu.VMEM((_K, 128), jnp.float32),
        pltpu.VMEM((_TSL,), jnp.float32),
        pltpu.VMEM_SHARED((_NPAD,), jnp.float32),
        pltpu.VMEM_SHARED((_NPAD,), jnp.float32),
        pltpu.SemaphoreType.DMA,
    ],
)(_sc_out_body)


def _tc_dinv_body(degp_ref, x_ref, dinv_ref, y_ref):
  d = lax.rsqrt(degp_ref[0] + degp_ref[1])
  dinv_ref[...] = d
  for c in range(3):
    y_ref[c] = x_ref[c] * d


def _tc_z_body(aggp_ref, dinv_ref, x_ref, w1_ref, b1_ref, g1_ref, be1_ref,
               w3_ref, z_ref, zd_ref):
  d = dinv_ref[...]
  a = [d * (aggp_ref[0, c] + aggp_ref[1, c]) + d * d * x_ref[c]
       for c in range(3)]
  nn = float(_N)
  s1 = [jnp.sum(a[c]) / nn for c in range(3)]
  w1r = [w1_ref[c] for c in range(3)]
  b1 = b1_ref[...]
  mean = s1[0] * w1r[0] + s1[1] * w1r[1] + s1[2] * w1r[2] + b1
  var = jnp.zeros((32,), jnp.float32)
  for k in range(3):
    for l in range(3):
      ckl = jnp.sum(a[k] * a[l]) / nn - s1[k] * s1[l]
      var = var + ckl * w1r[k] * w1r[l]
  istd = g1_ref[...] * lax.rsqrt(var + _EPS)
  wf = [w1r[c] * istd for c in range(3)]
  cf = (b1 - mean) * istd + be1_ref[...]
  w3 = w3_ref[...][:, 0]
  acc = jnp.zeros((_NPAD // 128, 128), jnp.float32)
  for j in range(32):
    h = a[0] * wf[0][j] + a[1] * wf[1][j] + a[2] * wf[2][j] + cf[j]
    acc = acc + jnp.maximum(h, 0.0) * w3[j]
  z_ref[...] = acc
  zd_ref[...] = acc * d


def _tc_fin_body(outp_ref, dinv_ref, z_ref, b3_ref, o_ref):
  d = dinv_ref[...]
  o_ref[...] = (d * (outp_ref[0] + outp_ref[1]) + d * d * z_ref[...]
                + b3_ref[0, 0])


def kernel(x, edge_index, edge_attr, W1, b1, gamma1, beta1, W3, b3):
  f32 = jnp.float32
  row = edge_index[0].astype(jnp.int32)
  col = edge_index[1].astype(jnp.int32)
  w = edge_attr.astype(f32)

  npad_e = _EPAD - _E
  pad_idx = (_N + (jnp.arange(npad_e, dtype=jnp.int32) % (_NPAD - _N)))
  row_p = jnp.concatenate([row, pad_idx]).reshape(_ROWS, 128)
  col_p = jnp.concatenate([col, pad_idx]).reshape(_ROWS, 128)
  w_p = jnp.concatenate([w, jnp.zeros((npad_e,), f32)]).reshape(_ROWS, 128)

  xt = jnp.pad(x.astype(f32), ((0, _NPAD - _N), (0, 0))).T
  x3d = xt.reshape(3, _NPAD // 128, 128)

  degp = _sc_deg(col_p, w_p)

  dinv2d, y3d = pl.pallas_call(
      _tc_dinv_body,
      out_shape=[
          jax.ShapeDtypeStruct((_NPAD // 128, 128), f32),
          jax.ShapeDtypeStruct((3, _NPAD // 128, 128), f32),
      ],
  )(degp.reshape(_NC, _NPAD // 128, 128), x3d)
  y0, y1, y2 = (y3d.reshape(3, _NPAD)[c] for c in range(3))

  aggp = _sc_agg(row_p, col_p, w_p, y0, y1, y2)

  z2d, zd2d = pl.pallas_call(
      _tc_z_body,
      out_shape=[
          jax.ShapeDtypeStruct((_NPAD // 128, 128), f32),
          jax.ShapeDtypeStruct((_NPAD // 128, 128), f32),
      ],
  )(aggp.reshape(_NC, 3, _NPAD // 128, 128), dinv2d, x3d, W1.astype(f32),
    b1.astype(f32), gamma1.astype(f32), beta1.astype(f32), W3.astype(f32))
  zd = zd2d.reshape(_NPAD)

  outp = _sc_out(row_p, col_p, w_p, zd)

  out2d = pl.pallas_call(
      _tc_fin_body,
      out_shape=jax.ShapeDtypeStruct((_NPAD // 128, 128), f32),
  )(outp.reshape(_NC, _NPAD // 128, 128), dinv2d, z2d,
    b3.astype(f32).reshape(1, 1))

  return out2d.reshape(_NPAD)[:_N, None]
